# Initial kernel scaffold; baseline (speedup 1.0000x reference)
import jax
import jax.numpy as jnp
from jax.experimental import pallas as pl


def kernel(features, object_labels):
    raise NotImplementedError("write your pallas kernel here")


# speedup vs baseline: 2.2116x; 2.2116x over previous
import functools

import jax
import jax.numpy as jnp
from jax import lax
from jax.experimental import pallas as pl

NCLS = 16


def _miner_kernel(feat_ref, labt_ref, hp_ref, hn_ref, dp_ref, dn_ref, va_ref,
                  *, bm: int, b: int):
    i = pl.program_id(0)

    def mode_labels(ol, n):
        best_cnt = jnp.sum((ol == 0).astype(jnp.int32), axis=0)
        best_lab = jnp.zeros((n,), dtype=jnp.int32)
        for c in range(1, NCLS):
            cnt = jnp.sum((ol == c).astype(jnp.int32), axis=0)
            take = cnt > best_cnt
            best_lab = jnp.where(take, jnp.int32(c), best_lab)
            best_cnt = jnp.where(take, cnt, best_cnt)
        return best_lab

    best_lab = mode_labels(labt_ref[...], b)
    lab_a = mode_labels(labt_ref[:, pl.ds(i * bm, bm)], bm)

    f_all = feat_ref[...]
    a = feat_ref[pl.ds(i * bm, bm), :]
    sq_all = jnp.sum(f_all * f_all, axis=1)
    sq_a = jnp.sum(a * a, axis=1)
    g = jax.lax.dot_general(a, f_all, (((1,), (1,)), ((), ())),
                            preferred_element_type=jnp.float32)
    d2 = sq_a[:, None] + sq_all[None, :] - 2.0 * g
    dd = jnp.maximum(d2, 1e-12)

    same = lab_a[:, None] == best_lab[None, :]
    cols = lax.broadcasted_iota(jnp.int32, (bm, b), 1)
    rows = lax.broadcasted_iota(jnp.int32, (bm, b), 0) + i * bm
    not_self = rows != cols

    neg_inf = jnp.float32(-jnp.inf)
    pos_inf = jnp.float32(jnp.inf)
    pos_d = jnp.where(same & not_self, dd, neg_inf)
    neg_d = jnp.where(same, pos_inf, dd)

    bp = jnp.max(pos_d, axis=1)
    hp = jnp.min(jnp.where(pos_d == bp[:, None], cols, jnp.int32(b)), axis=1)
    bn = jnp.min(neg_d, axis=1)
    hn = jnp.min(jnp.where(neg_d == bn[:, None], cols, jnp.int32(b)), axis=1)

    n_same = jnp.sum(same.astype(jnp.int32), axis=1)
    has_pos = n_same > 1
    has_neg = n_same < b

    d0 = jnp.sqrt(dd[:, 0])
    dp = jnp.where(has_pos, jnp.sqrt(jnp.maximum(bp, 1e-12)), d0)
    dn = jnp.where(has_neg, jnp.sqrt(bn), d0)

    hp_ref[0, 0, :] = hp
    hn_ref[0, 0, :] = hn
    dp_ref[0, 0, :] = dp
    dn_ref[0, 0, :] = dn
    va_ref[0, 0, :] = (has_pos & has_neg).astype(jnp.int32)


@functools.partial(jax.jit, static_argnames=("interpret",))
def kernel(features, object_labels, interpret=False):
    b, d = features.shape
    bm = 256
    nb = b // bm
    labt = object_labels.astype(jnp.int32).T

    out_specs = [pl.BlockSpec((1, 1, bm), lambda i: (i, 0, 0)) for _ in range(5)]
    out_shapes = [
        jax.ShapeDtypeStruct((nb, 1, bm), jnp.int32),
        jax.ShapeDtypeStruct((nb, 1, bm), jnp.int32),
        jax.ShapeDtypeStruct((nb, 1, bm), jnp.float32),
        jax.ShapeDtypeStruct((nb, 1, bm), jnp.float32),
        jax.ShapeDtypeStruct((nb, 1, bm), jnp.int32),
    ]
    hp, hn, dp, dn, va = pl.pallas_call(
        functools.partial(_miner_kernel, bm=bm, b=b),
        grid=(nb,),
        in_specs=[
            pl.BlockSpec((b, d), lambda i: (0, 0)),
            pl.BlockSpec(labt.shape, lambda i: (0, 0)),
        ],
        out_specs=out_specs,
        out_shape=out_shapes,
        interpret=interpret,
    )(features, labt)

    hp = hp.reshape(b)
    hn = hn.reshape(b)
    anchors = jnp.arange(b, dtype=hp.dtype)
    triplets = jnp.stack([anchors, hp, hn], axis=1)
    valid_mask = va.reshape(b).astype(bool)
    mined = jnp.stack([dp.reshape(b), dn.reshape(b)], axis=1)
    return triplets, valid_mask, mined


# speedup vs baseline: 2.7163x; 1.2282x over previous
import functools

import jax
import jax.numpy as jnp
from jax import lax
from jax.experimental import pallas as pl
from jax.experimental.pallas import tpu as pltpu

NCLS = 16


def _miner_kernel(feat_ref, labt_ref, hp_ref, hn_ref, dp_ref, dn_ref, va_ref,
                  lab_scr, sq_scr, *, bm: int, b: int):
    i = pl.program_id(0)

    @pl.when(i == 0)
    def _init():
        ol = labt_ref[...]
        best_cnt = jnp.sum((ol == 0).astype(jnp.int32), axis=0)
        best_lab = jnp.zeros((b,), dtype=jnp.int32)
        for c in range(1, NCLS):
            cnt = jnp.sum((ol == c).astype(jnp.int32), axis=0)
            take = cnt > best_cnt
            best_lab = jnp.where(take, jnp.int32(c), best_lab)
            best_cnt = jnp.where(take, cnt, best_cnt)
        lab_scr[0, :] = best_lab
        f_all = feat_ref[...]
        sq_scr[0, :] = jnp.sum(f_all * f_all, axis=1)

    lab_vec = lab_scr[0, :]
    sq_vec = sq_scr[0, :]
    lab_a = lab_scr[0, pl.ds(i * bm, bm)]
    sq_a = sq_scr[0, pl.ds(i * bm, bm)]

    a = feat_ref[pl.ds(i * bm, bm), :]
    g = lax.dot_general(a, feat_ref[...], (((1,), (1,)), ((), ())),
                        preferred_element_type=jnp.float32)
    e2 = sq_vec[None, :] - 2.0 * g
    e2c = jnp.maximum(e2, (1e-12 - sq_a)[:, None])

    same = lab_a[:, None] == lab_vec[None, :]
    cols = lax.broadcasted_iota(jnp.int32, (bm, b), 1)
    row_gid = lax.broadcasted_iota(jnp.int32, (bm, 1), 0) + i * bm
    not_self = cols != row_gid

    neg_inf = jnp.float32(-jnp.inf)
    pos_inf = jnp.float32(jnp.inf)
    pos_d = jnp.where(same & not_self, e2c, neg_inf)
    neg_d = jnp.where(same, pos_inf, e2c)

    bp = jnp.max(pos_d, axis=1)
    hp = jnp.min(jnp.where(pos_d == bp[:, None], cols, jnp.int32(b)), axis=1)
    bn = jnp.min(neg_d, axis=1)
    hn = jnp.min(jnp.where(neg_d == bn[:, None], cols, jnp.int32(b)), axis=1)

    has_pos = bp != neg_inf
    has_neg = bn != pos_inf

    d0 = jnp.sqrt(e2c[:, 0] + sq_a)
    dp = jnp.where(has_pos, jnp.sqrt(jnp.maximum(bp + sq_a, 1e-12)), d0)
    dn = jnp.where(has_neg, jnp.sqrt(bn + sq_a), d0)

    hp_ref[0, 0, :] = hp
    hn_ref[0, 0, :] = hn
    dp_ref[0, 0, :] = dp
    dn_ref[0, 0, :] = dn
    va_ref[0, 0, :] = (has_pos & has_neg).astype(jnp.int32)


@functools.partial(jax.jit, static_argnames=("interpret",))
def kernel(features, object_labels, interpret=False):
    b, d = features.shape
    bm = 256
    nb = b // bm
    labt = object_labels.astype(jnp.int32).T

    out_specs = [pl.BlockSpec((1, 1, bm), lambda i: (i, 0, 0)) for _ in range(5)]
    out_shapes = [
        jax.ShapeDtypeStruct((nb, 1, bm), jnp.int32),
        jax.ShapeDtypeStruct((nb, 1, bm), jnp.int32),
        jax.ShapeDtypeStruct((nb, 1, bm), jnp.float32),
        jax.ShapeDtypeStruct((nb, 1, bm), jnp.float32),
        jax.ShapeDtypeStruct((nb, 1, bm), jnp.int32),
    ]
    hp, hn, dp, dn, va = pl.pallas_call(
        functools.partial(_miner_kernel, bm=bm, b=b),
        grid=(nb,),
        in_specs=[
            pl.BlockSpec((b, d), lambda i: (0, 0)),
            pl.BlockSpec(labt.shape, lambda i: (0, 0)),
        ],
        out_specs=out_specs,
        out_shape=out_shapes,
        scratch_shapes=[
            pltpu.VMEM((1, b), jnp.int32),
            pltpu.VMEM((1, b), jnp.float32),
        ],
        interpret=interpret,
    )(features, labt)

    hp = hp.reshape(b)
    hn = hn.reshape(b)
    anchors = jnp.arange(b, dtype=hp.dtype)
    triplets = jnp.stack([anchors, hp, hn], axis=1)
    valid_mask = va.reshape(b).astype(bool)
    mined = jnp.stack([dp.reshape(b), dn.reshape(b)], axis=1)
    return triplets, valid_mask, mined


# speedup vs baseline: 2.7856x; 1.0255x over previous
import functools

import jax
import jax.numpy as jnp
from jax import lax
from jax.experimental import pallas as pl
from jax.experimental.pallas import tpu as pltpu

NCLS = 16


def _miner_kernel(feat_ref, labt_ref, hp_ref, hn_ref, dp_ref, dn_ref, va_ref,
                  lab_scr, sq_scr, *, bm: int, b: int):
    i = pl.program_id(0)

    @pl.when(i == 0)
    def _init():
        ol = labt_ref[...]
        best_cnt = jnp.sum((ol == 0).astype(jnp.int32), axis=0)
        best_lab = jnp.zeros((b,), dtype=jnp.int32)
        for c in range(1, NCLS):
            cnt = jnp.sum((ol == c).astype(jnp.int32), axis=0)
            take = cnt > best_cnt
            best_lab = jnp.where(take, jnp.int32(c), best_lab)
            best_cnt = jnp.where(take, cnt, best_cnt)
        lab_scr[0, :] = best_lab
        f_all = feat_ref[...]
        sq_scr[0, :] = jnp.sum(f_all * f_all, axis=1)

    lab_vec = lab_scr[0, :]
    sq_vec = sq_scr[0, :]
    lab_a = lab_scr[0, pl.ds(i * bm, bm)]
    sq_a = sq_scr[0, pl.ds(i * bm, bm)]

    a = feat_ref[pl.ds(i * bm, bm), :]
    g = lax.dot_general(a, feat_ref[...], (((1,), (1,)), ((), ())),
                        preferred_element_type=jnp.float32)
    e2 = sq_vec[None, :] - 2.0 * g
    e2c = jnp.maximum(e2, (1e-12 - sq_a)[:, None])

    same = lab_a[:, None] == lab_vec[None, :]
    cols = lax.broadcasted_iota(jnp.int32, (bm, b), 1)
    row_gid = lax.broadcasted_iota(jnp.int32, (bm, 1), 0) + i * bm
    not_self = cols != row_gid

    neg_inf = jnp.float32(-jnp.inf)
    pos_inf = jnp.float32(jnp.inf)
    pos_d = jnp.where(same & not_self, e2c, neg_inf)
    neg_d = jnp.where(same, pos_inf, e2c)

    bp = jnp.max(pos_d, axis=1)
    hp = jnp.argmax(pos_d, axis=1).astype(jnp.int32)
    bn = jnp.min(neg_d, axis=1)
    hn = jnp.argmin(neg_d, axis=1).astype(jnp.int32)

    has_pos = bp != neg_inf
    has_neg = bn != pos_inf
    hp = jnp.where(has_pos, hp, 0)
    hn = jnp.where(has_neg, hn, 0)

    d0 = jnp.sqrt(e2c[:, 0] + sq_a)
    dp = jnp.where(has_pos, jnp.sqrt(jnp.maximum(bp + sq_a, 1e-12)), d0)
    dn = jnp.where(has_neg, jnp.sqrt(bn + sq_a), d0)

    hp_ref[0, 0, :] = hp
    hn_ref[0, 0, :] = hn
    dp_ref[0, 0, :] = dp
    dn_ref[0, 0, :] = dn
    va_ref[0, 0, :] = (has_pos & has_neg).astype(jnp.int32)


@functools.partial(jax.jit, static_argnames=("interpret",))
def kernel(features, object_labels, interpret=False):
    b, d = features.shape
    bm = 256
    nb = b // bm
    labt = object_labels.astype(jnp.int32).T

    out_specs = [pl.BlockSpec((1, 1, bm), lambda i: (i, 0, 0)) for _ in range(5)]
    out_shapes = [
        jax.ShapeDtypeStruct((nb, 1, bm), jnp.int32),
        jax.ShapeDtypeStruct((nb, 1, bm), jnp.int32),
        jax.ShapeDtypeStruct((nb, 1, bm), jnp.float32),
        jax.ShapeDtypeStruct((nb, 1, bm), jnp.float32),
        jax.ShapeDtypeStruct((nb, 1, bm), jnp.int32),
    ]
    hp, hn, dp, dn, va = pl.pallas_call(
        functools.partial(_miner_kernel, bm=bm, b=b),
        grid=(nb,),
        in_specs=[
            pl.BlockSpec((b, d), lambda i: (0, 0)),
            pl.BlockSpec(labt.shape, lambda i: (0, 0)),
        ],
        out_specs=out_specs,
        out_shape=out_shapes,
        scratch_shapes=[
            pltpu.VMEM((1, b), jnp.int32),
            pltpu.VMEM((1, b), jnp.float32),
        ],
        interpret=interpret,
    )(features, labt)

    hp = hp.reshape(b)
    hn = hn.reshape(b)
    anchors = jnp.arange(b, dtype=hp.dtype)
    triplets = jnp.stack([anchors, hp, hn], axis=1)
    valid_mask = va.reshape(b).astype(bool)
    mined = jnp.stack([dp.reshape(b), dn.reshape(b)], axis=1)
    return triplets, valid_mask, mined
